# Initial kernel scaffold; baseline (speedup 1.0000x reference)
import jax
import jax.numpy as jnp
from jax.experimental import pallas as pl


def kernel(x, edge_index, edge_attr, batch, mask, params):
    raise NotImplementedError("write your pallas kernel here")


# speedup vs baseline: 1.5377x; 1.5377x over previous
import functools

import jax
import jax.numpy as jnp
from jax import lax
from jax.experimental import pallas as pl
from jax.experimental.pallas import tpu as pltpu

H = 128
G = 16
LAYERS = 3

_NEG = -1e9


def _relu(v):
    return jnp.maximum(v, 0.0)


def _dot(a, b):
    return jnp.dot(a, b, preferred_element_type=jnp.float32,
                   precision=lax.Precision.HIGHEST)


def _node_proc_body(x_ref, w_ref, b_ref, batch_ref, hn_ref, oh_ref,
                    ncnt_ref, cacc_ref):
    i = pl.program_id(0)

    @pl.when(i == 0)
    def _():
        cacc_ref[...] = jnp.zeros_like(cacc_ref)

    hn_ref[...] = _relu(_dot(x_ref[...], w_ref[...]) + b_ref[...])
    oh = (batch_ref[...] == lax.broadcasted_iota(jnp.int32, (1, G), 1)
          ).astype(jnp.float32)
    oh_ref[...] = oh
    cacc_ref[...] += jnp.sum(oh, axis=0, keepdims=True)

    @pl.when(i == pl.num_programs(0) - 1)
    def _():
        ncnt_ref[...] = jnp.maximum(cacc_ref[...], 1.0)


def _node_proc(x, wn, bn, batch2d, bn_blk):
    n = x.shape[0]
    grid = n // bn_blk
    return pl.pallas_call(
        _node_proc_body,
        grid=(grid,),
        in_specs=[
            pl.BlockSpec((bn_blk, x.shape[1]), lambda i: (i, 0)),
            pl.BlockSpec((x.shape[1], H), lambda i: (0, 0)),
            pl.BlockSpec((1, H), lambda i: (0, 0)),
            pl.BlockSpec((bn_blk, 1), lambda i: (i, 0)),
        ],
        out_specs=[
            pl.BlockSpec((bn_blk, H), lambda i: (i, 0)),
            pl.BlockSpec((bn_blk, G), lambda i: (i, 0)),
            pl.BlockSpec((1, G), lambda i: (0, 0)),
        ],
        out_shape=[
            jax.ShapeDtypeStruct((n, H), jnp.float32),
            jax.ShapeDtypeStruct((n, G), jnp.float32),
            jax.ShapeDtypeStruct((1, G), jnp.float32),
        ],
        scratch_shapes=[pltpu.VMEM((1, G), jnp.float32)],
    )(x, wn, bn, batch2d)


def _edge_proc_body(ea_ref, w_ref, b_ref, he_ref):
    he_ref[...] = _relu(_dot(ea_ref[...], w_ref[...]) + b_ref[...])


def _edge_proc(edge_attr, we, be, be_blk):
    e, f = edge_attr.shape
    return pl.pallas_call(
        _edge_proc_body,
        grid=(e // be_blk,),
        in_specs=[
            pl.BlockSpec((be_blk, f), lambda i: (i, 0)),
            pl.BlockSpec((f, H), lambda i: (0, 0)),
            pl.BlockSpec((1, H), lambda i: (0, 0)),
        ],
        out_specs=pl.BlockSpec((be_blk, H), lambda i: (i, 0)),
        out_shape=jax.ShapeDtypeStruct((e, H), jnp.float32),
    )(edge_attr, we, be)


def _tables_body(hn_ref, oh_ref, u_ref, w2_ref, w3_ref, w4_ref, bias_ref,
                 a_ref, b_ref):
    uw4 = _dot(u_ref[...], w4_ref[...])
    a_ref[...] = (_dot(hn_ref[...], w2_ref[...]) + _dot(oh_ref[...], uw4)
                  + bias_ref[...])
    b_ref[...] = _dot(hn_ref[...], w3_ref[...])


def _tables(hn, oh, u, w2, w3, w4, bias, bn_blk):
    n = hn.shape[0]
    return pl.pallas_call(
        _tables_body,
        grid=(n // bn_blk,),
        in_specs=[
            pl.BlockSpec((bn_blk, H), lambda i: (i, 0)),
            pl.BlockSpec((bn_blk, G), lambda i: (i, 0)),
            pl.BlockSpec((G, H), lambda i: (0, 0)),
            pl.BlockSpec((H, H), lambda i: (0, 0)),
            pl.BlockSpec((H, H), lambda i: (0, 0)),
            pl.BlockSpec((H, H), lambda i: (0, 0)),
            pl.BlockSpec((1, H), lambda i: (0, 0)),
        ],
        out_specs=[
            pl.BlockSpec((bn_blk, H), lambda i: (i, 0)),
            pl.BlockSpec((bn_blk, H), lambda i: (i, 0)),
        ],
        out_shape=[
            jax.ShapeDtypeStruct((n, H), jnp.float32),
            jax.ShapeDtypeStruct((n, H), jnp.float32),
        ],
    )(hn, oh, u, w2, w3, w4, bias)


def _edge_update_body(he_ref, msg_ref, ohe_ref, w1_ref,
                      heo_ref, esum_ref, ecnt_ref, acc_ref, cacc_ref):
    i = pl.program_id(0)

    @pl.when(i == 0)
    def _():
        acc_ref[...] = jnp.zeros_like(acc_ref)
        cacc_ref[...] = jnp.zeros_like(cacc_ref)

    h = _relu(_dot(he_ref[...], w1_ref[...]) + msg_ref[...])
    heo_ref[...] = h
    ohe = ohe_ref[...]
    acc_ref[...] += lax.dot_general(ohe, h, (((0,), (0,)), ((), ())),
                                    preferred_element_type=jnp.float32,
                                    precision=lax.Precision.HIGHEST)
    cacc_ref[...] += jnp.sum(ohe, axis=0, keepdims=True)

    @pl.when(i == pl.num_programs(0) - 1)
    def _():
        esum_ref[...] = acc_ref[...]
        ecnt_ref[...] = jnp.maximum(cacc_ref[...], 1.0)


def _edge_update(he, msg, ohe, w1, be_blk):
    e = he.shape[0]
    return pl.pallas_call(
        _edge_update_body,
        grid=(e // be_blk,),
        in_specs=[
            pl.BlockSpec((be_blk, H), lambda i: (i, 0)),
            pl.BlockSpec((be_blk, H), lambda i: (i, 0)),
            pl.BlockSpec((be_blk, G), lambda i: (i, 0)),
            pl.BlockSpec((H, H), lambda i: (0, 0)),
        ],
        out_specs=[
            pl.BlockSpec((be_blk, H), lambda i: (i, 0)),
            pl.BlockSpec((G, H), lambda i: (0, 0)),
            pl.BlockSpec((1, G), lambda i: (0, 0)),
        ],
        out_shape=[
            jax.ShapeDtypeStruct((e, H), jnp.float32),
            jax.ShapeDtypeStruct((G, H), jnp.float32),
            jax.ShapeDtypeStruct((1, G), jnp.float32),
        ],
        scratch_shapes=[pltpu.VMEM((G, H), jnp.float32),
                        pltpu.VMEM((1, G), jnp.float32)],
    )(he, msg, ohe, w1)


def _node_update_body(hn_ref, agg_ref, oh_ref, u_ref,
                      wv1_ref, wv2_ref, wv3_ref, bv_ref,
                      hno_ref, nsum_ref, acc_ref):
    i = pl.program_id(0)

    @pl.when(i == 0)
    def _():
        acc_ref[...] = jnp.zeros_like(acc_ref)

    uw3 = _dot(u_ref[...], wv3_ref[...])
    h = _relu(_dot(hn_ref[...], wv1_ref[...]) + _dot(agg_ref[...], wv2_ref[...])
              + _dot(oh_ref[...], uw3) + bv_ref[...])
    hno_ref[...] = h
    acc_ref[...] += lax.dot_general(oh_ref[...], h, (((0,), (0,)), ((), ())),
                                    preferred_element_type=jnp.float32,
                                    precision=lax.Precision.HIGHEST)

    @pl.when(i == pl.num_programs(0) - 1)
    def _():
        nsum_ref[...] = acc_ref[...]


def _node_update(hn, agg, oh, u, wv1, wv2, wv3, bv, bn_blk):
    n = hn.shape[0]
    return pl.pallas_call(
        _node_update_body,
        grid=(n // bn_blk,),
        in_specs=[
            pl.BlockSpec((bn_blk, H), lambda i: (i, 0)),
            pl.BlockSpec((bn_blk, H), lambda i: (i, 0)),
            pl.BlockSpec((bn_blk, G), lambda i: (i, 0)),
            pl.BlockSpec((G, H), lambda i: (0, 0)),
            pl.BlockSpec((H, H), lambda i: (0, 0)),
            pl.BlockSpec((H, H), lambda i: (0, 0)),
            pl.BlockSpec((H, H), lambda i: (0, 0)),
            pl.BlockSpec((1, H), lambda i: (0, 0)),
        ],
        out_specs=[
            pl.BlockSpec((bn_blk, H), lambda i: (i, 0)),
            pl.BlockSpec((G, H), lambda i: (0, 0)),
        ],
        out_shape=[
            jax.ShapeDtypeStruct((n, H), jnp.float32),
            jax.ShapeDtypeStruct((G, H), jnp.float32),
        ],
        scratch_shapes=[pltpu.VMEM((G, H), jnp.float32)],
    )(hn, agg, oh, u, wv1, wv2, wv3, bv)


def _global_body(u_ref, nsum_ref, ncnt_ref, esum_ref, ecnt_ref,
                 wu1_ref, wu2_ref, wu3_ref, bu_ref, uo_ref):
    n_mean = nsum_ref[...] / ncnt_ref[...]
    e_mean = esum_ref[...] / ecnt_ref[...]
    uo_ref[...] = _relu(_dot(u_ref[...], wu1_ref[...])
                        + _dot(n_mean, wu2_ref[...])
                        + _dot(e_mean, wu3_ref[...]) + bu_ref[...])


def _global_update(u, nsum, ncnt, esum, ecnt, wu1, wu2, wu3, bu):
    return pl.pallas_call(
        _global_body,
        out_shape=jax.ShapeDtypeStruct((G, H), jnp.float32),
    )(u, nsum, ncnt, esum, ecnt, wu1, wu2, wu3, bu)


def _logits_body(hn_ref, wa_ref, ba_ref, out_ref):
    z = _dot(hn_ref[...], wa_ref[...]) + ba_ref[...]
    out_ref[...] = 1.0 / (1.0 + jnp.exp(-z))


def _logits(hn, wa, ba, bn_blk):
    n = hn.shape[0]
    return pl.pallas_call(
        _logits_body,
        grid=(n // bn_blk,),
        in_specs=[
            pl.BlockSpec((bn_blk, H), lambda i: (i, 0)),
            pl.BlockSpec((H, 1), lambda i: (0, 0)),
            pl.BlockSpec((1, 1), lambda i: (0, 0)),
        ],
        out_specs=pl.BlockSpec((bn_blk, 1), lambda i: (i, 0)),
        out_shape=jax.ShapeDtypeStruct((n, 1), jnp.float32),
    )(hn, wa, ba)


def _head_body(lg_ref, mask_ref, gum_ref, u_ref, wc_ref, bc_ref,
               act_ref, lp_ref, ent_ref, val_ref):
    lm = jnp.where(mask_ref[...], lg_ref[...], _NEG)
    mx = jnp.max(lm, axis=-1, keepdims=True)
    ex = jnp.exp(lm - mx)
    se = jnp.sum(ex, axis=-1, keepdims=True)
    lse = jnp.log(se) + mx
    logp = lm - lse
    p = ex / se
    ent_ref[...] = -jnp.sum(p * logp, axis=-1, keepdims=True)
    pert = lm + gum_ref[...]
    acts = jnp.argmax(pert, axis=-1)[:, None]
    act_ref[...] = acts.astype(jnp.int32)
    lanes = lax.broadcasted_iota(jnp.int32, lm.shape, 1)
    sel = lanes == acts
    lp_ref[...] = jnp.sum(jnp.where(sel, logp, 0.0), axis=-1, keepdims=True)
    val_ref[...] = _dot(u_ref[...], wc_ref[...]) + bc_ref[...]


def _head(lg, maskp, gum, u, wc, bc):
    return pl.pallas_call(
        _head_body,
        out_shape=[
            jax.ShapeDtypeStruct((G, 1), jnp.int32),
            jax.ShapeDtypeStruct((G, 1), jnp.float32),
            jax.ShapeDtypeStruct((G, 1), jnp.float32),
            jax.ShapeDtypeStruct((G, 1), jnp.float32),
        ],
    )(lg, maskp, gum, u, wc, bc)


def _gather_msg(a_tab, b_tab, src, dst):
    return jnp.take(a_tab, src, axis=0) + jnp.take(b_tab, dst, axis=0)


def _scatter_agg(he, dst, n):
    return jax.ops.segment_sum(he, dst, num_segments=n)


def kernel(x, edge_index, edge_attr, batch, mask, params):
    n, node_f = x.shape
    e = edge_attr.shape[0]
    src = edge_index[0]
    dst = edge_index[1]

    bn_blk = 2000
    be_blk = 8000

    batch2d = batch.astype(jnp.int32).reshape(n, 1)
    bn_b = params['bn'].reshape(1, H)
    be_b = params['be'].reshape(1, H)

    h_n, oh, ncnt = _node_proc(x, params['Wn'], bn_b, batch2d, bn_blk)
    ncnt = ncnt.reshape(G, 1)
    h_e = _edge_proc(edge_attr, params['We'], be_b, be_blk)
    oh_e = jnp.take(oh, src, axis=0)

    u = jnp.tile(params['init_u'], (G, 1))

    for l in range(LAYERS):
        we_l = params['We_%d' % l]
        w1, w2, w3, w4 = (we_l[0:H], we_l[H:2 * H], we_l[2 * H:3 * H],
                          we_l[3 * H:4 * H])
        bias = params['be_%d' % l].reshape(1, H)
        a_tab, b_tab = _tables(h_n, oh, u, w2, w3, w4, bias, bn_blk)
        msg = _gather_msg(a_tab, b_tab, src, dst)
        h_e, esum, ecnt = _edge_update(h_e, msg, oh_e, w1, be_blk)
        ecnt = ecnt.reshape(G, 1)
        agg = _scatter_agg(h_e, dst, n)
        wv_l = params['Wv_%d' % l]
        wv1, wv2, wv3 = wv_l[0:H], wv_l[H:2 * H], wv_l[2 * H:3 * H]
        bv = params['bv_%d' % l].reshape(1, H)
        h_n, nsum = _node_update(h_n, agg, oh, u, wv1, wv2, wv3, bv, bn_blk)
        wu_l = params['Wu_%d' % l]
        wu1, wu2, wu3 = wu_l[0:H], wu_l[H:2 * H], wu_l[2 * H:3 * H]
        bu = params['bu_%d' % l].reshape(1, H)
        u = _global_update(u, nsum, ncnt, esum, ecnt, wu1, wu2, wu3, bu)

    lg = _logits(h_n, params['Wa'], params['ba'].reshape(1, 1), bn_blk)

    w = n // G
    pad = (-w) % 128
    lg2 = jnp.pad(lg.reshape(G, w), ((0, 0), (0, pad)))
    maskp = jnp.pad(mask, ((0, 0), (0, pad)))
    gum = jax.random.gumbel(jax.random.key(42), (G, w), jnp.float32)
    gum = jnp.pad(gum, ((0, 0), (0, pad)))

    acts, lps, ent, val = _head(lg2, maskp, gum, u,
                                params['Wc'], params['bc'].reshape(1, 1))
    return (acts[:, 0], lps[:, 0], ent[:, 0], val)


# speedup vs baseline: 1.9804x; 1.2879x over previous
import functools

import jax
import jax.numpy as jnp
from jax import lax
from jax.experimental import pallas as pl
from jax.experimental.pallas import tpu as pltpu
from jax.experimental.pallas import tpu_sc as plsc

H = 128
G = 16
LAYERS = 3

_NEG = -1e9


def _relu(v):
    return jnp.maximum(v, 0.0)


def _dot(a, b):
    return jnp.dot(a, b, preferred_element_type=jnp.float32,
                   precision=lax.Precision.HIGHEST)


def _node_proc_body(x_ref, w_ref, b_ref, batch_ref, hn_ref, oh_ref,
                    ncnt_ref, cacc_ref):
    i = pl.program_id(0)

    @pl.when(i == 0)
    def _():
        cacc_ref[...] = jnp.zeros_like(cacc_ref)

    hn_ref[...] = _relu(_dot(x_ref[...], w_ref[...]) + b_ref[...])
    oh = (batch_ref[...] == lax.broadcasted_iota(jnp.int32, (1, G), 1)
          ).astype(jnp.float32)
    oh_ref[...] = oh
    cacc_ref[...] += jnp.sum(oh, axis=0, keepdims=True)

    @pl.when(i == pl.num_programs(0) - 1)
    def _():
        ncnt_ref[...] = jnp.maximum(cacc_ref[...], 1.0)


def _node_proc(x, wn, bn, batch2d, bn_blk):
    n = x.shape[0]
    grid = n // bn_blk
    return pl.pallas_call(
        _node_proc_body,
        grid=(grid,),
        in_specs=[
            pl.BlockSpec((bn_blk, x.shape[1]), lambda i: (i, 0)),
            pl.BlockSpec((x.shape[1], H), lambda i: (0, 0)),
            pl.BlockSpec((1, H), lambda i: (0, 0)),
            pl.BlockSpec((bn_blk, 1), lambda i: (i, 0)),
        ],
        out_specs=[
            pl.BlockSpec((bn_blk, H), lambda i: (i, 0)),
            pl.BlockSpec((bn_blk, G), lambda i: (i, 0)),
            pl.BlockSpec((1, G), lambda i: (0, 0)),
        ],
        out_shape=[
            jax.ShapeDtypeStruct((n, H), jnp.float32),
            jax.ShapeDtypeStruct((n, G), jnp.float32),
            jax.ShapeDtypeStruct((1, G), jnp.float32),
        ],
        scratch_shapes=[pltpu.VMEM((1, G), jnp.float32)],
    )(x, wn, bn, batch2d)


def _edge_proc_body(ea_ref, w_ref, b_ref, he_ref):
    he_ref[...] = _relu(_dot(ea_ref[...], w_ref[...]) + b_ref[...])


def _edge_proc(edge_attr, we, be, be_blk):
    e, f = edge_attr.shape
    return pl.pallas_call(
        _edge_proc_body,
        grid=(e // be_blk,),
        in_specs=[
            pl.BlockSpec((be_blk, f), lambda i: (i, 0)),
            pl.BlockSpec((f, H), lambda i: (0, 0)),
            pl.BlockSpec((1, H), lambda i: (0, 0)),
        ],
        out_specs=pl.BlockSpec((be_blk, H), lambda i: (i, 0)),
        out_shape=jax.ShapeDtypeStruct((e, H), jnp.float32),
    )(edge_attr, we, be)


def _tables_body(hn_ref, oh_ref, u_ref, w2_ref, w3_ref, w4_ref, bias_ref,
                 a_ref, b_ref):
    uw4 = _dot(u_ref[...], w4_ref[...])
    a_ref[...] = (_dot(hn_ref[...], w2_ref[...]) + _dot(oh_ref[...], uw4)
                  + bias_ref[...])
    b_ref[...] = _dot(hn_ref[...], w3_ref[...])


def _tables(hn, oh, u, w2, w3, w4, bias, bn_blk):
    n = hn.shape[0]
    return pl.pallas_call(
        _tables_body,
        grid=(n // bn_blk,),
        in_specs=[
            pl.BlockSpec((bn_blk, H), lambda i: (i, 0)),
            pl.BlockSpec((bn_blk, G), lambda i: (i, 0)),
            pl.BlockSpec((G, H), lambda i: (0, 0)),
            pl.BlockSpec((H, H), lambda i: (0, 0)),
            pl.BlockSpec((H, H), lambda i: (0, 0)),
            pl.BlockSpec((H, H), lambda i: (0, 0)),
            pl.BlockSpec((1, H), lambda i: (0, 0)),
        ],
        out_specs=[
            pl.BlockSpec((bn_blk, H), lambda i: (i, 0)),
            pl.BlockSpec((bn_blk, H), lambda i: (i, 0)),
        ],
        out_shape=[
            jax.ShapeDtypeStruct((n, H), jnp.float32),
            jax.ShapeDtypeStruct((n, H), jnp.float32),
        ],
    )(hn, oh, u, w2, w3, w4, bias)


def _edge_update_body(he_ref, msg_ref, ohe_ref, w1_ref,
                      heo_ref, esum_ref, ecnt_ref, acc_ref, cacc_ref):
    i = pl.program_id(0)

    @pl.when(i == 0)
    def _():
        acc_ref[...] = jnp.zeros_like(acc_ref)
        cacc_ref[...] = jnp.zeros_like(cacc_ref)

    h = _relu(_dot(he_ref[...], w1_ref[...]) + msg_ref[...])
    heo_ref[...] = h
    ohe = ohe_ref[...]
    acc_ref[...] += lax.dot_general(ohe, h, (((0,), (0,)), ((), ())),
                                    preferred_element_type=jnp.float32,
                                    precision=lax.Precision.HIGHEST)
    cacc_ref[...] += jnp.sum(ohe, axis=0, keepdims=True)

    @pl.when(i == pl.num_programs(0) - 1)
    def _():
        esum_ref[...] = acc_ref[...]
        ecnt_ref[...] = jnp.maximum(cacc_ref[...], 1.0)


def _edge_update(he, msg, ohe, w1, be_blk):
    e = he.shape[0]
    return pl.pallas_call(
        _edge_update_body,
        grid=(e // be_blk,),
        in_specs=[
            pl.BlockSpec((be_blk, H), lambda i: (i, 0)),
            pl.BlockSpec((be_blk, H), lambda i: (i, 0)),
            pl.BlockSpec((be_blk, G), lambda i: (i, 0)),
            pl.BlockSpec((H, H), lambda i: (0, 0)),
        ],
        out_specs=[
            pl.BlockSpec((be_blk, H), lambda i: (i, 0)),
            pl.BlockSpec((G, H), lambda i: (0, 0)),
            pl.BlockSpec((1, G), lambda i: (0, 0)),
        ],
        out_shape=[
            jax.ShapeDtypeStruct((e, H), jnp.float32),
            jax.ShapeDtypeStruct((G, H), jnp.float32),
            jax.ShapeDtypeStruct((1, G), jnp.float32),
        ],
        scratch_shapes=[pltpu.VMEM((G, H), jnp.float32),
                        pltpu.VMEM((1, G), jnp.float32)],
    )(he, msg, ohe, w1)


def _node_update_body(hn_ref, agg_ref, agg2_ref, oh_ref, u_ref,
                      wv1_ref, wv2_ref, wv3_ref, bv_ref,
                      hno_ref, nsum_ref, acc_ref):
    i = pl.program_id(0)

    @pl.when(i == 0)
    def _():
        acc_ref[...] = jnp.zeros_like(acc_ref)

    uw3 = _dot(u_ref[...], wv3_ref[...])
    agg = agg_ref[...] + agg2_ref[...]
    h = _relu(_dot(hn_ref[...], wv1_ref[...]) + _dot(agg, wv2_ref[...])
              + _dot(oh_ref[...], uw3) + bv_ref[...])
    hno_ref[...] = h
    acc_ref[...] += lax.dot_general(oh_ref[...], h, (((0,), (0,)), ((), ())),
                                    preferred_element_type=jnp.float32,
                                    precision=lax.Precision.HIGHEST)

    @pl.when(i == pl.num_programs(0) - 1)
    def _():
        nsum_ref[...] = acc_ref[...]


def _node_update(hn, agg, agg2, oh, u, wv1, wv2, wv3, bv, bn_blk):
    n = hn.shape[0]
    return pl.pallas_call(
        _node_update_body,
        grid=(n // bn_blk,),
        in_specs=[
            pl.BlockSpec((bn_blk, H), lambda i: (i, 0)),
            pl.BlockSpec((bn_blk, H), lambda i: (i, 0)),
            pl.BlockSpec((bn_blk, H), lambda i: (i, 0)),
            pl.BlockSpec((bn_blk, G), lambda i: (i, 0)),
            pl.BlockSpec((G, H), lambda i: (0, 0)),
            pl.BlockSpec((H, H), lambda i: (0, 0)),
            pl.BlockSpec((H, H), lambda i: (0, 0)),
            pl.BlockSpec((H, H), lambda i: (0, 0)),
            pl.BlockSpec((1, H), lambda i: (0, 0)),
        ],
        out_specs=[
            pl.BlockSpec((bn_blk, H), lambda i: (i, 0)),
            pl.BlockSpec((G, H), lambda i: (0, 0)),
        ],
        out_shape=[
            jax.ShapeDtypeStruct((n, H), jnp.float32),
            jax.ShapeDtypeStruct((G, H), jnp.float32),
        ],
        scratch_shapes=[pltpu.VMEM((G, H), jnp.float32)],
    )(hn, agg, agg2, oh, u, wv1, wv2, wv3, bv)


def _global_body(u_ref, nsum_ref, ncnt_ref, esum_ref, ecnt_ref,
                 wu1_ref, wu2_ref, wu3_ref, bu_ref, uo_ref):
    n_mean = nsum_ref[...] / ncnt_ref[...]
    e_mean = esum_ref[...] / ecnt_ref[...]
    uo_ref[...] = _relu(_dot(u_ref[...], wu1_ref[...])
                        + _dot(n_mean, wu2_ref[...])
                        + _dot(e_mean, wu3_ref[...]) + bu_ref[...])


def _global_update(u, nsum, ncnt, esum, ecnt, wu1, wu2, wu3, bu):
    return pl.pallas_call(
        _global_body,
        out_shape=jax.ShapeDtypeStruct((G, H), jnp.float32),
    )(u, nsum, ncnt, esum, ecnt, wu1, wu2, wu3, bu)


def _logits_body(hn_ref, wa_ref, ba_ref, out_ref):
    z = _dot(hn_ref[...], wa_ref[...]) + ba_ref[...]
    out_ref[...] = 1.0 / (1.0 + jnp.exp(-z))


def _logits(hn, wa, ba, bn_blk):
    n = hn.shape[0]
    return pl.pallas_call(
        _logits_body,
        grid=(n // bn_blk,),
        in_specs=[
            pl.BlockSpec((bn_blk, H), lambda i: (i, 0)),
            pl.BlockSpec((H, 1), lambda i: (0, 0)),
            pl.BlockSpec((1, 1), lambda i: (0, 0)),
        ],
        out_specs=pl.BlockSpec((bn_blk, 1), lambda i: (i, 0)),
        out_shape=jax.ShapeDtypeStruct((n, 1), jnp.float32),
    )(hn, wa, ba)


def _head_body(lg_ref, mask_ref, gum_ref, u_ref, wc_ref, bc_ref,
               act_ref, lp_ref, ent_ref, val_ref):
    lm = jnp.where(mask_ref[...], lg_ref[...], _NEG)
    mx = jnp.max(lm, axis=-1, keepdims=True)
    ex = jnp.exp(lm - mx)
    se = jnp.sum(ex, axis=-1, keepdims=True)
    lse = jnp.log(se) + mx
    logp = lm - lse
    p = ex / se
    ent_ref[...] = -jnp.sum(p * logp, axis=-1, keepdims=True)
    pert = lm + gum_ref[...]
    acts = jnp.argmax(pert, axis=-1)[:, None]
    act_ref[...] = acts.astype(jnp.int32)
    lanes = lax.broadcasted_iota(jnp.int32, lm.shape, 1)
    sel = lanes == acts
    lp_ref[...] = jnp.sum(jnp.where(sel, logp, 0.0), axis=-1, keepdims=True)
    val_ref[...] = _dot(u_ref[...], wc_ref[...]) + bc_ref[...]


def _head(lg, maskp, gum, u, wc, bc):
    return pl.pallas_call(
        _head_body,
        out_shape=[
            jax.ShapeDtypeStruct((G, 1), jnp.int32),
            jax.ShapeDtypeStruct((G, 1), jnp.float32),
            jax.ShapeDtypeStruct((G, 1), jnp.float32),
            jax.ShapeDtypeStruct((G, 1), jnp.float32),
        ],
    )(lg, maskp, gum, u, wc, bc)


def _gather_msg(a_tab, b_tab, src, dst):
    return jnp.take(a_tab, src, axis=0) + jnp.take(b_tab, dst, axis=0)


_SC_NC = 2
_SC_NS = 16
_SC_CH = 200


def _make_sc_scatter(e, n):
    nw = _SC_NC * _SC_NS
    epw = e // nw
    nchunk = epw // _SC_CH
    rps = (n // _SC_NS) & ~7
    rem = n - _SC_NS * rps
    wbs = []
    off = 0
    while off < rps:
        wbs.append((off, min(_SC_CH, rps - off)))
        off += min(_SC_CH, rps - off)
    mesh = plsc.VectorSubcoreMesh(core_axis_name="c", subcore_axis_name="s")

    @functools.partial(
        pl.kernel, mesh=mesh,
        out_type=jax.ShapeDtypeStruct((_SC_NC * n, H), jnp.float32),
        scratch_types=[
            pltpu.VMEM((_SC_CH,), jnp.int32),
            pltpu.VMEM((_SC_CH, H), jnp.float32),
            pltpu.VMEM_SHARED((n, H), jnp.float32),
        ],
    )
    def sc_scatter(he_hbm, dst_hbm, zeros_hbm, out_hbm, idx_v, rows_v, shared):
        c = lax.axis_index("c")
        s = lax.axis_index("s")
        wid = s * _SC_NC + c
        base = wid * epw
        r0 = s * rps

        pltpu.sync_copy(zeros_hbm.at[pl.ds(r0, rps)], shared.at[pl.ds(r0, rps)])
        if rem > 0:
            @pl.when(s == _SC_NS - 1)
            def _():
                pltpu.sync_copy(zeros_hbm.at[pl.ds(_SC_NS * rps, rem)],
                                shared.at[pl.ds(_SC_NS * rps, rem)])
        plsc.subcore_barrier()

        @pl.loop(0, nchunk)
        def _(k):
            off = base + k * _SC_CH
            pltpu.sync_copy(dst_hbm.at[pl.ds(off, _SC_CH)], idx_v)
            pltpu.sync_copy(he_hbm.at[pl.ds(off, _SC_CH)], rows_v)
            pltpu.sync_copy(rows_v, shared.at[idx_v], add=True)

        plsc.subcore_barrier()

        for wo, wl in wbs:
            pltpu.sync_copy(shared.at[pl.ds(r0 + wo, wl)],
                            rows_v.at[pl.ds(0, wl)])
            pltpu.sync_copy(rows_v.at[pl.ds(0, wl)],
                            out_hbm.at[pl.ds(c * n + r0 + wo, wl)])
        if rem > 0:
            @pl.when(s == _SC_NS - 1)
            def _():
                t0 = _SC_NS * rps
                pltpu.sync_copy(shared.at[pl.ds(t0, rem)],
                                rows_v.at[pl.ds(0, rem)])
                pltpu.sync_copy(rows_v.at[pl.ds(0, rem)],
                                out_hbm.at[pl.ds(c * n + t0, rem)])

    return sc_scatter


def _scatter_agg(he, dst, zeros, sc_fn, n):
    parts = sc_fn(he, dst, zeros)
    return parts[:n], parts[n:]


def kernel(x, edge_index, edge_attr, batch, mask, params):
    n, node_f = x.shape
    e = edge_attr.shape[0]
    src = edge_index[0]
    dst = edge_index[1]

    bn_blk = 2000
    be_blk = 8000

    batch2d = batch.astype(jnp.int32).reshape(n, 1)
    bn_b = params['bn'].reshape(1, H)
    be_b = params['be'].reshape(1, H)

    h_n, oh, ncnt = _node_proc(x, params['Wn'], bn_b, batch2d, bn_blk)
    ncnt = ncnt.reshape(G, 1)
    h_e = _edge_proc(edge_attr, params['We'], be_b, be_blk)
    oh_e = jnp.take(oh, src, axis=0)

    u = jnp.tile(params['init_u'], (G, 1))

    sc_fn = _make_sc_scatter(e, n)
    zeros = jnp.zeros((n, H), jnp.float32)

    for l in range(LAYERS):
        we_l = params['We_%d' % l]
        w1, w2, w3, w4 = (we_l[0:H], we_l[H:2 * H], we_l[2 * H:3 * H],
                          we_l[3 * H:4 * H])
        bias = params['be_%d' % l].reshape(1, H)
        a_tab, b_tab = _tables(h_n, oh, u, w2, w3, w4, bias, bn_blk)
        msg = _gather_msg(a_tab, b_tab, src, dst)
        h_e, esum, ecnt = _edge_update(h_e, msg, oh_e, w1, be_blk)
        ecnt = ecnt.reshape(G, 1)
        agg_a, agg_b = _scatter_agg(h_e, dst, zeros, sc_fn, n)
        wv_l = params['Wv_%d' % l]
        wv1, wv2, wv3 = wv_l[0:H], wv_l[H:2 * H], wv_l[2 * H:3 * H]
        bv = params['bv_%d' % l].reshape(1, H)
        h_n, nsum = _node_update(h_n, agg_a, agg_b, oh, u, wv1, wv2, wv3, bv,
                                 bn_blk)
        wu_l = params['Wu_%d' % l]
        wu1, wu2, wu3 = wu_l[0:H], wu_l[H:2 * H], wu_l[2 * H:3 * H]
        bu = params['bu_%d' % l].reshape(1, H)
        u = _global_update(u, nsum, ncnt, esum, ecnt, wu1, wu2, wu3, bu)

    lg = _logits(h_n, params['Wa'], params['ba'].reshape(1, 1), bn_blk)

    w = n // G
    pad = (-w) % 128
    lg2 = jnp.pad(lg.reshape(G, w), ((0, 0), (0, pad)))
    maskp = jnp.pad(mask, ((0, 0), (0, pad)))
    gum = jax.random.gumbel(jax.random.key(42), (G, w), jnp.float32)
    gum = jnp.pad(gum, ((0, 0), (0, pad)))

    acts, lps, ent, val = _head(lg2, maskp, gum, u,
                                params['Wc'], params['bc'].reshape(1, 1))
    return (acts[:, 0], lps[:, 0], ent[:, 0], val)


# speedup vs baseline: 1.9812x; 1.0004x over previous
import functools

import jax
import jax.numpy as jnp
from jax import lax
from jax.experimental import pallas as pl
from jax.experimental.pallas import tpu as pltpu
from jax.experimental.pallas import tpu_sc as plsc

H = 128
G = 16
LAYERS = 3

_NEG = -1e9


def _relu(v):
    return jnp.maximum(v, 0.0)


def _dot(a, b):
    return jnp.dot(a, b, preferred_element_type=jnp.float32,
                   precision=lax.Precision.HIGHEST)


def _node_proc_body(x_ref, w_ref, b_ref, batch_ref, hn_ref, oh_ref,
                    ncnt_ref, cacc_ref):
    i = pl.program_id(0)

    @pl.when(i == 0)
    def _():
        cacc_ref[...] = jnp.zeros_like(cacc_ref)

    hn_ref[...] = _relu(_dot(x_ref[...], w_ref[...]) + b_ref[...])
    oh = (batch_ref[...] == lax.broadcasted_iota(jnp.int32, (1, G), 1)
          ).astype(jnp.float32)
    oh_ref[...] = oh
    cacc_ref[...] += jnp.sum(oh, axis=0, keepdims=True)

    @pl.when(i == pl.num_programs(0) - 1)
    def _():
        ncnt_ref[...] = jnp.maximum(cacc_ref[...], 1.0)


def _node_proc(x, wn, bn, batch2d, bn_blk):
    n = x.shape[0]
    grid = n // bn_blk
    return pl.pallas_call(
        _node_proc_body,
        grid=(grid,),
        in_specs=[
            pl.BlockSpec((bn_blk, x.shape[1]), lambda i: (i, 0)),
            pl.BlockSpec((x.shape[1], H), lambda i: (0, 0)),
            pl.BlockSpec((1, H), lambda i: (0, 0)),
            pl.BlockSpec((bn_blk, 1), lambda i: (i, 0)),
        ],
        out_specs=[
            pl.BlockSpec((bn_blk, H), lambda i: (i, 0)),
            pl.BlockSpec((bn_blk, G), lambda i: (i, 0)),
            pl.BlockSpec((1, G), lambda i: (0, 0)),
        ],
        out_shape=[
            jax.ShapeDtypeStruct((n, H), jnp.float32),
            jax.ShapeDtypeStruct((n, G), jnp.float32),
            jax.ShapeDtypeStruct((1, G), jnp.float32),
        ],
        scratch_shapes=[pltpu.VMEM((1, G), jnp.float32)],
    )(x, wn, bn, batch2d)


def _edge_proc_body(ea_ref, w_ref, b_ref, he_ref):
    he_ref[...] = _relu(_dot(ea_ref[...], w_ref[...]) + b_ref[...])


def _edge_proc(edge_attr, we, be, be_blk):
    e, f = edge_attr.shape
    return pl.pallas_call(
        _edge_proc_body,
        grid=(e // be_blk,),
        in_specs=[
            pl.BlockSpec((be_blk, f), lambda i: (i, 0)),
            pl.BlockSpec((f, H), lambda i: (0, 0)),
            pl.BlockSpec((1, H), lambda i: (0, 0)),
        ],
        out_specs=pl.BlockSpec((be_blk, H), lambda i: (i, 0)),
        out_shape=jax.ShapeDtypeStruct((e, H), jnp.float32),
    )(edge_attr, we, be)


def _tables_body(hn_ref, oh_ref, u_ref, w2_ref, w3_ref, w4_ref, bias_ref,
                 a_ref, b_ref):
    uw4 = _dot(u_ref[...], w4_ref[...])
    a_ref[...] = (_dot(hn_ref[...], w2_ref[...]) + _dot(oh_ref[...], uw4)
                  + bias_ref[...])
    b_ref[...] = _dot(hn_ref[...], w3_ref[...])


def _tables(hn, oh, u, w2, w3, w4, bias, bn_blk):
    n = hn.shape[0]
    return pl.pallas_call(
        _tables_body,
        grid=(n // bn_blk,),
        in_specs=[
            pl.BlockSpec((bn_blk, H), lambda i: (i, 0)),
            pl.BlockSpec((bn_blk, G), lambda i: (i, 0)),
            pl.BlockSpec((G, H), lambda i: (0, 0)),
            pl.BlockSpec((H, H), lambda i: (0, 0)),
            pl.BlockSpec((H, H), lambda i: (0, 0)),
            pl.BlockSpec((H, H), lambda i: (0, 0)),
            pl.BlockSpec((1, H), lambda i: (0, 0)),
        ],
        out_specs=[
            pl.BlockSpec((bn_blk, H), lambda i: (i, 0)),
            pl.BlockSpec((bn_blk, H), lambda i: (i, 0)),
        ],
        out_shape=[
            jax.ShapeDtypeStruct((n, H), jnp.float32),
            jax.ShapeDtypeStruct((n, H), jnp.float32),
        ],
    )(hn, oh, u, w2, w3, w4, bias)


def _edge_update_body(he_ref, msg_ref, ohe_ref, w1_ref,
                      heo_ref, esum_ref, ecnt_ref, acc_ref, cacc_ref):
    i = pl.program_id(0)

    @pl.when(i == 0)
    def _():
        acc_ref[...] = jnp.zeros_like(acc_ref)
        cacc_ref[...] = jnp.zeros_like(cacc_ref)

    h = _relu(_dot(he_ref[...], w1_ref[...]) + msg_ref[...])
    heo_ref[...] = h
    ohe = ohe_ref[...]
    acc_ref[...] += lax.dot_general(ohe, h, (((0,), (0,)), ((), ())),
                                    preferred_element_type=jnp.float32,
                                    precision=lax.Precision.HIGHEST)
    cacc_ref[...] += jnp.sum(ohe, axis=0, keepdims=True)

    @pl.when(i == pl.num_programs(0) - 1)
    def _():
        esum_ref[...] = acc_ref[...]
        ecnt_ref[...] = jnp.maximum(cacc_ref[...], 1.0)


def _edge_update(he, msg, ohe, w1, be_blk):
    e = he.shape[0]
    return pl.pallas_call(
        _edge_update_body,
        grid=(e // be_blk,),
        in_specs=[
            pl.BlockSpec((be_blk, H), lambda i: (i, 0)),
            pl.BlockSpec((be_blk, H), lambda i: (i, 0)),
            pl.BlockSpec((be_blk, G), lambda i: (i, 0)),
            pl.BlockSpec((H, H), lambda i: (0, 0)),
        ],
        out_specs=[
            pl.BlockSpec((be_blk, H), lambda i: (i, 0)),
            pl.BlockSpec((G, H), lambda i: (0, 0)),
            pl.BlockSpec((1, G), lambda i: (0, 0)),
        ],
        out_shape=[
            jax.ShapeDtypeStruct((e, H), jnp.float32),
            jax.ShapeDtypeStruct((G, H), jnp.float32),
            jax.ShapeDtypeStruct((1, G), jnp.float32),
        ],
        scratch_shapes=[pltpu.VMEM((G, H), jnp.float32),
                        pltpu.VMEM((1, G), jnp.float32)],
    )(he, msg, ohe, w1)


def _node_update_body(hn_ref, agg_ref, agg2_ref, oh_ref, u_ref,
                      wv1_ref, wv2_ref, wv3_ref, bv_ref,
                      hno_ref, nsum_ref, acc_ref):
    i = pl.program_id(0)

    @pl.when(i == 0)
    def _():
        acc_ref[...] = jnp.zeros_like(acc_ref)

    uw3 = _dot(u_ref[...], wv3_ref[...])
    agg = agg_ref[...] + agg2_ref[...]
    h = _relu(_dot(hn_ref[...], wv1_ref[...]) + _dot(agg, wv2_ref[...])
              + _dot(oh_ref[...], uw3) + bv_ref[...])
    hno_ref[...] = h
    acc_ref[...] += lax.dot_general(oh_ref[...], h, (((0,), (0,)), ((), ())),
                                    preferred_element_type=jnp.float32,
                                    precision=lax.Precision.HIGHEST)

    @pl.when(i == pl.num_programs(0) - 1)
    def _():
        nsum_ref[...] = acc_ref[...]


def _node_update(hn, agg, agg2, oh, u, wv1, wv2, wv3, bv, bn_blk):
    n = hn.shape[0]
    return pl.pallas_call(
        _node_update_body,
        grid=(n // bn_blk,),
        in_specs=[
            pl.BlockSpec((bn_blk, H), lambda i: (i, 0)),
            pl.BlockSpec((bn_blk, H), lambda i: (i, 0)),
            pl.BlockSpec((bn_blk, H), lambda i: (i, 0)),
            pl.BlockSpec((bn_blk, G), lambda i: (i, 0)),
            pl.BlockSpec((G, H), lambda i: (0, 0)),
            pl.BlockSpec((H, H), lambda i: (0, 0)),
            pl.BlockSpec((H, H), lambda i: (0, 0)),
            pl.BlockSpec((H, H), lambda i: (0, 0)),
            pl.BlockSpec((1, H), lambda i: (0, 0)),
        ],
        out_specs=[
            pl.BlockSpec((bn_blk, H), lambda i: (i, 0)),
            pl.BlockSpec((G, H), lambda i: (0, 0)),
        ],
        out_shape=[
            jax.ShapeDtypeStruct((n, H), jnp.float32),
            jax.ShapeDtypeStruct((G, H), jnp.float32),
        ],
        scratch_shapes=[pltpu.VMEM((G, H), jnp.float32)],
    )(hn, agg, agg2, oh, u, wv1, wv2, wv3, bv)


def _global_body(u_ref, nsum_ref, ncnt_ref, esum_ref, ecnt_ref,
                 wu1_ref, wu2_ref, wu3_ref, bu_ref, uo_ref):
    n_mean = nsum_ref[...] / ncnt_ref[...]
    e_mean = esum_ref[...] / ecnt_ref[...]
    uo_ref[...] = _relu(_dot(u_ref[...], wu1_ref[...])
                        + _dot(n_mean, wu2_ref[...])
                        + _dot(e_mean, wu3_ref[...]) + bu_ref[...])


def _global_update(u, nsum, ncnt, esum, ecnt, wu1, wu2, wu3, bu):
    return pl.pallas_call(
        _global_body,
        out_shape=jax.ShapeDtypeStruct((G, H), jnp.float32),
    )(u, nsum, ncnt, esum, ecnt, wu1, wu2, wu3, bu)


def _logits_body(hn_ref, wa_ref, ba_ref, out_ref):
    z = _dot(hn_ref[...], wa_ref[...]) + ba_ref[...]
    out_ref[...] = 1.0 / (1.0 + jnp.exp(-z))


def _logits(hn, wa, ba, bn_blk):
    n = hn.shape[0]
    return pl.pallas_call(
        _logits_body,
        grid=(n // bn_blk,),
        in_specs=[
            pl.BlockSpec((bn_blk, H), lambda i: (i, 0)),
            pl.BlockSpec((H, 1), lambda i: (0, 0)),
            pl.BlockSpec((1, 1), lambda i: (0, 0)),
        ],
        out_specs=pl.BlockSpec((bn_blk, 1), lambda i: (i, 0)),
        out_shape=jax.ShapeDtypeStruct((n, 1), jnp.float32),
    )(hn, wa, ba)


def _head_body(lg_ref, mask_ref, gum_ref, u_ref, wc_ref, bc_ref,
               act_ref, lp_ref, ent_ref, val_ref):
    lm = jnp.where(mask_ref[...], lg_ref[...], _NEG)
    mx = jnp.max(lm, axis=-1, keepdims=True)
    ex = jnp.exp(lm - mx)
    se = jnp.sum(ex, axis=-1, keepdims=True)
    lse = jnp.log(se) + mx
    logp = lm - lse
    p = ex / se
    ent_ref[...] = -jnp.sum(p * logp, axis=-1, keepdims=True)
    pert = lm + gum_ref[...]
    acts = jnp.argmax(pert, axis=-1)[:, None]
    act_ref[...] = acts.astype(jnp.int32)
    lanes = lax.broadcasted_iota(jnp.int32, lm.shape, 1)
    sel = lanes == acts
    lp_ref[...] = jnp.sum(jnp.where(sel, logp, 0.0), axis=-1, keepdims=True)
    val_ref[...] = _dot(u_ref[...], wc_ref[...]) + bc_ref[...]


def _head(lg, maskp, gum, u, wc, bc):
    return pl.pallas_call(
        _head_body,
        out_shape=[
            jax.ShapeDtypeStruct((G, 1), jnp.int32),
            jax.ShapeDtypeStruct((G, 1), jnp.float32),
            jax.ShapeDtypeStruct((G, 1), jnp.float32),
            jax.ShapeDtypeStruct((G, 1), jnp.float32),
        ],
    )(lg, maskp, gum, u, wc, bc)


def _gather_msg(a_tab, b_tab, src, dst):
    return jnp.take(a_tab, src, axis=0) + jnp.take(b_tab, dst, axis=0)


_SC_NC = 2
_SC_NS = 16
_SC_CH = 200


def _make_sc_scatter(e, n):
    nw = _SC_NC * _SC_NS
    epw = e // nw
    nchunk = epw // _SC_CH
    rps = (n // _SC_NS) & ~7
    rem = n - _SC_NS * rps
    wbs = []
    off = 0
    while off < rps:
        wbs.append((off, min(_SC_CH, rps - off)))
        off += min(_SC_CH, rps - off)
    mesh = plsc.VectorSubcoreMesh(core_axis_name="c", subcore_axis_name="s")

    @functools.partial(
        pl.kernel, mesh=mesh,
        out_type=jax.ShapeDtypeStruct((_SC_NC * n, H), jnp.float32),
        scratch_types=[
            pltpu.VMEM((_SC_CH,), jnp.int32),
            pltpu.VMEM((_SC_CH, H), jnp.float32),
            pltpu.VMEM_SHARED((n, H), jnp.float32),
        ],
    )
    def sc_scatter(he_hbm, dst_hbm, zeros_hbm, out_hbm, idx_v, rows_v, shared):
        c = lax.axis_index("c")
        s = lax.axis_index("s")
        wid = s * _SC_NC + c
        base = wid * epw
        r0 = s * rps

        pltpu.sync_copy(zeros_hbm.at[pl.ds(r0, rps)], shared.at[pl.ds(r0, rps)])
        if rem > 0:
            @pl.when(s == _SC_NS - 1)
            def _():
                pltpu.sync_copy(zeros_hbm.at[pl.ds(_SC_NS * rps, rem)],
                                shared.at[pl.ds(_SC_NS * rps, rem)])
        plsc.subcore_barrier()

        @pl.loop(0, nchunk)
        def _(k):
            off = base + k * _SC_CH
            pltpu.sync_copy(dst_hbm.at[pl.ds(off, _SC_CH)], idx_v)
            pltpu.sync_copy(he_hbm.at[pl.ds(off, _SC_CH)], rows_v)
            pltpu.sync_copy(rows_v, shared.at[idx_v], add=True)

        plsc.subcore_barrier()

        for wo, wl in wbs:
            pltpu.sync_copy(shared.at[pl.ds(r0 + wo, wl)],
                            rows_v.at[pl.ds(0, wl)])
            pltpu.sync_copy(rows_v.at[pl.ds(0, wl)],
                            out_hbm.at[pl.ds(c * n + r0 + wo, wl)])
        if rem > 0:
            @pl.when(s == _SC_NS - 1)
            def _():
                t0 = _SC_NS * rps
                pltpu.sync_copy(shared.at[pl.ds(t0, rem)],
                                rows_v.at[pl.ds(0, rem)])
                pltpu.sync_copy(rows_v.at[pl.ds(0, rem)],
                                out_hbm.at[pl.ds(c * n + t0, rem)])

    return sc_scatter


def _scatter_agg(he, dst, zeros, sc_fn, n):
    parts = sc_fn(he, dst, zeros)
    return parts[:n], parts[n:]


def kernel(x, edge_index, edge_attr, batch, mask, params):
    n, node_f = x.shape
    e = edge_attr.shape[0]
    src = edge_index[0]
    dst = edge_index[1]

    bn_blk = 2000
    be_blk = 10000

    batch2d = batch.astype(jnp.int32).reshape(n, 1)
    bn_b = params['bn'].reshape(1, H)
    be_b = params['be'].reshape(1, H)

    h_n, oh, ncnt = _node_proc(x, params['Wn'], bn_b, batch2d, bn_blk)
    ncnt = ncnt.reshape(G, 1)
    h_e = _edge_proc(edge_attr, params['We'], be_b, be_blk)
    oh_e = jnp.take(oh, src, axis=0)

    u = jnp.tile(params['init_u'], (G, 1))

    sc_fn = _make_sc_scatter(e, n)
    zeros = jnp.zeros((n, H), jnp.float32)

    for l in range(LAYERS):
        we_l = params['We_%d' % l]
        w1, w2, w3, w4 = (we_l[0:H], we_l[H:2 * H], we_l[2 * H:3 * H],
                          we_l[3 * H:4 * H])
        bias = params['be_%d' % l].reshape(1, H)
        a_tab, b_tab = _tables(h_n, oh, u, w2, w3, w4, bias, bn_blk)
        msg = _gather_msg(a_tab, b_tab, src, dst)
        h_e, esum, ecnt = _edge_update(h_e, msg, oh_e, w1, be_blk)
        ecnt = ecnt.reshape(G, 1)
        agg_a, agg_b = _scatter_agg(h_e, dst, zeros, sc_fn, n)
        wv_l = params['Wv_%d' % l]
        wv1, wv2, wv3 = wv_l[0:H], wv_l[H:2 * H], wv_l[2 * H:3 * H]
        bv = params['bv_%d' % l].reshape(1, H)
        h_n, nsum = _node_update(h_n, agg_a, agg_b, oh, u, wv1, wv2, wv3, bv,
                                 bn_blk)
        wu_l = params['Wu_%d' % l]
        wu1, wu2, wu3 = wu_l[0:H], wu_l[H:2 * H], wu_l[2 * H:3 * H]
        bu = params['bu_%d' % l].reshape(1, H)
        u = _global_update(u, nsum, ncnt, esum, ecnt, wu1, wu2, wu3, bu)

    lg = _logits(h_n, params['Wa'], params['ba'].reshape(1, 1), bn_blk)

    w = n // G
    pad = (-w) % 128
    lg2 = jnp.pad(lg.reshape(G, w), ((0, 0), (0, pad)))
    maskp = jnp.pad(mask, ((0, 0), (0, pad)))
    gum = jax.random.gumbel(jax.random.key(42), (G, w), jnp.float32)
    gum = jnp.pad(gum, ((0, 0), (0, pad)))

    acts, lps, ent, val = _head(lg2, maskp, gum, u,
                                params['Wc'], params['bc'].reshape(1, 1))
    return (acts[:, 0], lps[:, 0], ent[:, 0], val)
